# Initial kernel scaffold; baseline (speedup 1.0000x reference)
import functools

import jax
import jax.numpy as jnp
from jax import lax
from jax.experimental import pallas as pl
from jax.experimental.pallas import tpu as pltpu
from jax.experimental.pallas import tpu_sc as plsc

B, K, H, W = 32, 17, 128, 128
HW = H * W
_FI = tuple(0 if k == 0 else (k + 1 if k % 2 == 1 else k - 1) for k in range(K))

_CHUNKS_PER_ROW = W // 16


def _rev_plane(vin, vout, negate):
    def body(h, carry):
        base = h * W
        for j in range(_CHUNKS_PER_ROW):
            src = base + (_CHUNKS_PER_ROW - 1 - j) * 16
            v = lax.rev(vin[pl.ds(src, 16)], (0,))
            if negate:
                v = -v
            vout[pl.ds(base + j * 16, 16)] = v
        return carry

    lax.fori_loop(0, H, body, 0)


def _sc_flip(conf, reg, scale):
    mesh = plsc.VectorSubcoreMesh(core_axis_name="c", subcore_axis_name="s")

    @functools.partial(
        pl.kernel,
        mesh=mesh,
        out_type=(
            jax.ShapeDtypeStruct((B, K, HW), jnp.float32),
            jax.ShapeDtypeStruct((B, K, 2, HW), jnp.float32),
            jax.ShapeDtypeStruct((B, K, HW), jnp.float32),
        ),
        scratch_types=[
            pltpu.VMEM((HW,), jnp.float32),
            pltpu.VMEM((HW,), jnp.float32),
        ],
    )
    def k(conf_in, reg_in, scale_in, conf_out, reg_out, scale_out, bin_, bout):
        w = lax.axis_index("s") * 2 + lax.axis_index("c")
        for src_ref, dst_ref in ((conf_in, conf_out), (scale_in, scale_out)):
            for kk in range(K):
                pltpu.sync_copy(src_ref.at[w, _FI[kk]], bin_)
                _rev_plane(bin_, bout, False)
                pltpu.sync_copy(bout, dst_ref.at[w, kk])
        for c in range(2):
            for kk in range(K):
                pltpu.sync_copy(reg_in.at[w, _FI[kk], c], bin_)
                _rev_plane(bin_, bout, c == 0)
                pltpu.sync_copy(bout, reg_out.at[w, kk, c])

    return k(conf, reg, scale)


def kernel(field_conf, field_reg, field_scale):
    conf = field_conf.reshape(B, K, HW)
    reg = field_reg.reshape(B, K, 2, HW)
    scale = field_scale.reshape(B, K, HW)
    oc, orr, os = _sc_flip(conf, reg, scale)
    return (
        oc.reshape(B, K, H, W),
        orr.reshape(B, K, 2, H, W),
        os.reshape(B, K, H, W),
    )


# speedup vs baseline: 8.0146x; 8.0146x over previous
import functools

import jax
import jax.numpy as jnp
from jax import lax
from jax.experimental import pallas as pl
from jax.experimental.pallas import tpu as pltpu
from jax.experimental.pallas import tpu_sc as plsc

B, K, H, W = 32, 17, 128, 128
HW = H * W
_FI = tuple(0 if k == 0 else (k + 1 if k % 2 == 1 else k - 1) for k in range(K))

_CHUNKS_PER_ROW = W // 16


def _rev_plane(vin, vout, negate):
    def body(h, carry):
        base = h * W
        for j in range(_CHUNKS_PER_ROW):
            src = base + (_CHUNKS_PER_ROW - 1 - j) * 16
            v = lax.rev(vin[pl.ds(src, 16)], (0,))
            if negate:
                v = -v
            vout[pl.ds(base + j * 16, 16)] = v
        return carry

    lax.fori_loop(0, H, body, 0)


def _sc_flip(conf, reg, scale):
    mesh = plsc.VectorSubcoreMesh(core_axis_name="c", subcore_axis_name="s")

    @functools.partial(
        pl.kernel,
        mesh=mesh,
        out_type=(
            jax.ShapeDtypeStruct((B, K, HW), jnp.float32),
            jax.ShapeDtypeStruct((B, K, 2, HW), jnp.float32),
            jax.ShapeDtypeStruct((B, K, HW), jnp.float32),
        ),
        scratch_types=[
            pltpu.VMEM((HW,), jnp.float32),
            pltpu.VMEM((HW,), jnp.float32),
        ],
        compiler_params=pltpu.CompilerParams(use_tc_tiling_on_sc=False),
    )
    def k(conf_in, reg_in, scale_in, conf_out, reg_out, scale_out, bin_, bout):
        w = lax.axis_index("s") * 2 + lax.axis_index("c")
        for src_ref, dst_ref in ((conf_in, conf_out), (scale_in, scale_out)):
            for kk in range(K):
                pltpu.sync_copy(src_ref.at[w, _FI[kk]], bin_)
                _rev_plane(bin_, bout, False)
                pltpu.sync_copy(bout, dst_ref.at[w, kk])
        for c in range(2):
            for kk in range(K):
                pltpu.sync_copy(reg_in.at[w, _FI[kk], c], bin_)
                _rev_plane(bin_, bout, c == 0)
                pltpu.sync_copy(bout, reg_out.at[w, kk, c])

    return k(conf, reg, scale)


def kernel(field_conf, field_reg, field_scale):
    conf = field_conf.reshape(B, K, HW)
    reg = field_reg.reshape(B, K, 2, HW)
    scale = field_scale.reshape(B, K, HW)
    oc, orr, os = _sc_flip(conf, reg, scale)
    return (
        oc.reshape(B, K, H, W),
        orr.reshape(B, K, 2, H, W),
        os.reshape(B, K, H, W),
    )


# speedup vs baseline: 13.7637x; 1.7173x over previous
import functools

import jax
import jax.numpy as jnp
from jax import lax
from jax.experimental import pallas as pl
from jax.experimental.pallas import tpu as pltpu
from jax.experimental.pallas import tpu_sc as plsc

B, K, H, W = 32, 17, 128, 128
HW = H * W
_FI = tuple(0 if k == 0 else (k + 1 if k % 2 == 1 else k - 1) for k in range(K))

_CHUNKS_PER_ROW = W // 16


def _rev_plane(vin, vout, negate):
    def body(h, carry):
        base = h * W
        for j in range(_CHUNKS_PER_ROW):
            src = base + (_CHUNKS_PER_ROW - 1 - j) * 16
            v = lax.rev(vin[pl.ds(src, 16)], (0,))
            if negate:
                v = -v
            vout[pl.ds(base + j * 16, 16)] = v
        return carry

    lax.fori_loop(0, H, body, 0)


def _sc_flip(conf, reg, scale):
    mesh = plsc.VectorSubcoreMesh(core_axis_name="c", subcore_axis_name="s")

    @functools.partial(
        pl.kernel,
        mesh=mesh,
        out_type=(
            jax.ShapeDtypeStruct((B, K, HW), jnp.float32),
            jax.ShapeDtypeStruct((B, K, 2, HW), jnp.float32),
            jax.ShapeDtypeStruct((B, K, HW), jnp.float32),
        ),
        scratch_types=[
            pltpu.VMEM((HW,), jnp.float32),
            pltpu.VMEM((HW,), jnp.float32),
            pltpu.VMEM((HW,), jnp.float32),
            pltpu.VMEM((HW,), jnp.float32),
            pltpu.SemaphoreType.DMA,
            pltpu.SemaphoreType.DMA,
            pltpu.SemaphoreType.DMA,
            pltpu.SemaphoreType.DMA,
        ],
        compiler_params=pltpu.CompilerParams(use_tc_tiling_on_sc=False),
    )
    def k(conf_in, reg_in, scale_in, conf_out, reg_out, scale_out,
          bin0, bin1, bout0, bout1, isem0, isem1, osem0, osem1):
        w = lax.axis_index("s") * 2 + lax.axis_index("c")
        bins, bouts = (bin0, bin1), (bout0, bout1)
        isems, osems = (isem0, isem1), (osem0, osem1)

        planes = []
        for src_ref, dst_ref in ((conf_in, conf_out), (scale_in, scale_out)):
            for kk in range(K):
                planes.append((src_ref.at[w, _FI[kk]], dst_ref.at[w, kk], False))
        for c in range(2):
            for kk in range(K):
                planes.append(
                    (reg_in.at[w, _FI[kk], c], reg_out.at[w, kk, c], c == 0))
        n = len(planes)

        copy_in = [None] * n
        copy_out = [None] * n
        copy_in[0] = pltpu.async_copy(planes[0][0], bins[0], isems[0])
        copy_in[1] = pltpu.async_copy(planes[1][0], bins[1], isems[1])
        for i in range(n):
            s = i % 2
            copy_in[i].wait()
            if i >= 2:
                copy_out[i - 2].wait()
            _rev_plane(bins[s], bouts[s], planes[i][2])
            copy_out[i] = pltpu.async_copy(bouts[s], planes[i][1], osems[s])
            if i + 2 < n:
                copy_in[i + 2] = pltpu.async_copy(
                    planes[i + 2][0], bins[s], isems[s])
        copy_out[n - 2].wait()
        copy_out[n - 1].wait()

    return k(conf, reg, scale)


def kernel(field_conf, field_reg, field_scale):
    conf = field_conf.reshape(B, K, HW)
    reg = field_reg.reshape(B, K, 2, HW)
    scale = field_scale.reshape(B, K, HW)
    oc, orr, os = _sc_flip(conf, reg, scale)
    return (
        oc.reshape(B, K, H, W),
        orr.reshape(B, K, 2, H, W),
        os.reshape(B, K, H, W),
    )
